# baseline (speedup 1.0000x reference)
import functools

import jax
import jax.numpy as jnp
from jax import lax
from jax.experimental import pallas as pl
from jax.experimental.pallas import tpu as pltpu
from jax.experimental.pallas import tpu_sc as plsc

_B = 4096
_SEQ = 64
_N_EMB = 26
_TOTAL_INPUT = 52
_VOCAB = 100000
_DIM = 64
_LANES = 16
_COLS = _DIM // _LANES

_NC = 2
_NS = 16
_NW = _NC * _NS
_BPW = _B // _NW
_IDX_PW = _BPW * _N_EMB
_CB = 4
_CIDX = _CB * _N_EMB
_NCHUNK = _BPW // _CB


def _emb_body(tab_hbm, idx_hbm, out_hbm, idx_v, rows_v, out_v, sem0, sem1):
    wid = lax.axis_index("s") * _NC + lax.axis_index("c")
    base = wid * _IDX_PW

    pltpu.sync_copy(idx_hbm.at[pl.ds(base, _IDX_PW)], idx_v)

    def xform(p, carry):
        s = pl.multiple_of(p * _LANES, _LANES)
        pos = lax.iota(jnp.int32, _LANES) + s
        idx_v[pl.ds(s, _LANES)] = (
            idx_v[pl.ds(s, _LANES)] + lax.rem(pos, _N_EMB) * _VOCAB
        )
        return carry

    lax.fori_loop(0, _IDX_PW // _LANES, xform, 0)

    def gather(c, buf, sem):
        return pltpu.async_copy(
            tab_hbm.at[idx_v.at[pl.ds(c * _CIDX, _CIDX)]],
            rows_v.at[buf],
            sem,
        )

    def wait(c, buf, sem):
        pltpu.make_async_copy(
            tab_hbm.at[idx_v.at[pl.ds(c * _CIDX, _CIDX)]],
            rows_v.at[buf],
            sem,
        ).wait()

    def accum(c, buf):
        for bi in range(_CB):
            def jbody(j, accs, _bi=bi):
                r = _bi * _N_EMB + j
                return tuple(
                    accs[t] + rows_v[buf, r, pl.ds(t * _LANES, _LANES)]
                    for t in range(_COLS)
                )

            accs = lax.fori_loop(
                0, _N_EMB, jbody,
                tuple(jnp.zeros((_LANES,), jnp.float32) for _ in range(_COLS)),
            )
            row = c * _CB + bi
            for t in range(_COLS):
                out_v[row, pl.ds(t * _LANES, _LANES)] = accs[t]

    gather(0, 0, sem0)
    gather(1, 1, sem1)

    def outer(i, carry):
        c0 = 2 * i
        wait(c0, 0, sem0)
        accum(c0, 0)

        @pl.when(c0 + 2 < _NCHUNK)
        def _():
            gather(c0 + 2, 0, sem0)

        c1 = c0 + 1
        wait(c1, 1, sem1)
        accum(c1, 1)

        @pl.when(c1 + 2 < _NCHUNK)
        def _():
            gather(c1 + 2, 1, sem1)

        return carry

    lax.fori_loop(0, _NCHUNK // 2, outer, 0)

    pltpu.sync_copy(out_v, out_hbm.at[pl.ds(wid * _BPW, _BPW)])


@jax.jit
def _embedding_sum(flat_tables, flat_idx):
    mesh = plsc.VectorSubcoreMesh(core_axis_name="c", subcore_axis_name="s")
    return pl.kernel(
        _emb_body,
        out_type=jax.ShapeDtypeStruct((_B, _DIM), jnp.float32),
        mesh=mesh,
        compiler_params=pltpu.CompilerParams(use_tc_tiling_on_sc=False),
        scratch_types=[
            pltpu.VMEM((_IDX_PW,), jnp.int32),
            pltpu.VMEM((2, _CIDX, _DIM), jnp.float32),
            pltpu.VMEM((_BPW, _DIM), jnp.float32),
            pltpu.SemaphoreType.DMA,
            pltpu.SemaphoreType.DMA,
        ],
    )(flat_tables, flat_idx)


def kernel(input, mask_key, tables):
    numeric = input[:, :, _N_EMB:].astype(jnp.float32)
    past_seq = numeric[:, :_TOTAL_INPUT, :]
    future_seq = numeric[:, _TOTAL_INPUT:, :]
    flat_tables = tables.reshape(_N_EMB * _VOCAB, _DIM)
    flat_idx = input[:, 0, :_N_EMB].reshape(-1)
    embedded_output = _embedding_sum(flat_tables, flat_idx)
    return (past_seq, future_seq, embedded_output)
